# baseline (speedup 1.0000x reference)
import functools

import jax
import jax.numpy as jnp
import numpy as np
from jax import lax
from jax.experimental import pallas as pl
from jax.experimental.pallas import tpu as pltpu
from jax.experimental.pallas import tpu_sc as plsc

N_NODES = 10000
N_EDGES = 320000
D_FEAT = 128
D_HALF = D_FEAT // 2

NUM_SUBCORES = 16
CHUNK = 128
CHUNKS_PER_TILE = 160
EDGES_PER_TILE = CHUNK * CHUNKS_PER_TILE
N_EDGES_PAD = EDGES_PER_TILE * NUM_SUBCORES
N_NODES_PAD = 10240
ROWS_PER_TILE = N_NODES_PAD // NUM_SUBCORES
X_ROWS_PER_TILE = N_NODES // NUM_SUBCORES
NBUF = 2
PHASES = 2
PCH = CHUNKS_PER_TILE // PHASES

_PERM = np.empty((D_FEAT,), dtype=np.int32)
for _blk in range(D_FEAT // 32):
    for _p in range(16):
        _PERM[_blk * 32 + 2 * _p] = _blk * 32 + _p
        _PERM[_blk * 32 + 2 * _p + 1] = _blk * 32 + 16 + _p

_mesh = plsc.VectorSubcoreMesh(core_axis_name="c", subcore_axis_name="s")


@functools.partial(
    pl.kernel,
    out_type=jax.ShapeDtypeStruct((N_NODES, D_FEAT), jnp.float32),
    mesh=_mesh,
    compiler_params=pltpu.CompilerParams(use_tc_tiling_on_sc=False,
                                         needs_layout_passes=False),
    scratch_types=[
        pltpu.VMEM((PCH, CHUNK), jnp.int32),
        pltpu.VMEM((PCH, CHUNK), jnp.int32),
        pltpu.VMEM((PCH, CHUNK), jnp.float32),
        pltpu.VMEM((CHUNK, D_HALF), jnp.bfloat16),
        pltpu.VMEM((CHUNK, D_HALF), jnp.bfloat16),
        pltpu.VMEM((CHUNK, D_HALF), jnp.float32),
        pltpu.VMEM((CHUNK, D_HALF), jnp.float32),
        pltpu.VMEM_SHARED((N_NODES, D_HALF), jnp.bfloat16),
        pltpu.VMEM_SHARED((N_NODES_PAD, D_HALF), jnp.float32),
        pltpu.SemaphoreType.DMA,
        pltpu.SemaphoreType.DMA,
        pltpu.SemaphoreType.DMA,
        pltpu.SemaphoreType.DMA,
    ],
)
def _spmm_sc(xb_h, col3_h, row3_h, val3_h, out_h,
             colv, rowv, valv, gbuf0, gbuf1, sbuf0, sbuf1, xs, acc,
             gsem0, gsem1, ssem0, ssem1):
    c = lax.axis_index("c")
    s = lax.axis_index("s")
    gbuf = (gbuf0, gbuf1)
    sbuf = (sbuf0, sbuf1)
    gsem = (gsem0, gsem1)
    ssem = (ssem0, ssem1)

    xsl = pl.ds(s * X_ROWS_PER_TILE, X_ROWS_PER_TILE)
    pltpu.async_copy(xb_h.at[xsl, pl.ds(c * D_HALF, D_HALF)], xs.at[xsl],
                     gsem0)

    def zero_body(i, carry):
        for f in range(D_HALF // 16):
            sbuf0[i, pl.ds(f * 16, 16)] = jnp.zeros((16,), jnp.float32)
        return carry
    lax.fori_loop(0, CHUNK, zero_body, 0)
    for i in range(ROWS_PER_TILE // CHUNK):
        pltpu.sync_copy(
            sbuf0, acc.at[pl.ds(s * ROWS_PER_TILE + i * CHUNK, CHUNK)])
    pltpu.make_async_copy(xb_h.at[xsl, pl.ds(c * D_HALF, D_HALF)],
                          xs.at[xsl], gsem0).wait()
    plsc.subcore_barrier()

    def gather_start(u, b):
        pltpu.async_copy(xs.at[colv.at[u]], gbuf[b], gsem[b])

    def gather_wait(u, b):
        pltpu.make_async_copy(xs.at[colv.at[u]], gbuf[b], gsem[b]).wait()

    def scatter_start(u, b):
        pltpu.async_copy(sbuf[b], acc.at[rowv.at[u]], ssem[b], add=True)

    def scatter_wait(u, b):
        pltpu.make_async_copy(sbuf[b], acc.at[rowv.at[u]], ssem[b]).wait()

    for h in range(PHASES):
        k_lo = h * PCH
        pltpu.sync_copy(col3_h.at[s, pl.ds(k_lo, PCH)], colv)
        pltpu.sync_copy(row3_h.at[s, pl.ds(k_lo, PCH)], rowv)
        pltpu.sync_copy(val3_h.at[s, pl.ds(k_lo, PCH)], valv)

        for b in range(NBUF):
            gather_start(b, b)

        def outer_body(o, carry):
            for b in range(NBUF):
                u = o * NBUF + b
                gather_wait(u, b)

                @pl.when(u >= NBUF)
                def _():
                    scatter_wait(u, b)

                def scale_body(g, inner):
                    vv = valv[u, pl.ds(g * 16, 16)]
                    for j in range(16):
                        e = g * 16 + j
                        v = vv[j]
                        for o2 in range(2):
                            t = gbuf[b][e, pl.ds(o2 * 32, 32)]
                            lo, hi = plsc.unpack(
                                t, format=plsc.PackFormat.INTERLEAVED)
                            sbuf[b][e, pl.ds(o2 * 32, 16)] = lo * v
                            sbuf[b][e, pl.ds(o2 * 32 + 16, 16)] = hi * v
                    return inner
                lax.fori_loop(0, CHUNK // 16, scale_body, 0)

                scatter_start(u, b)

                un = u + NBUF

                @pl.when(un < PCH)
                def _():
                    gather_start(un, b)
            return carry

        lax.fori_loop(0, PCH // NBUF, outer_body, 0)

        for b in range(NBUF):
            scatter_wait(0, b)

    plsc.subcore_barrier()

    csl = pl.ds(c * D_HALF, D_HALF)
    for i in range(ROWS_PER_TILE // CHUNK):
        base = s * ROWS_PER_TILE + i * CHUNK

        @pl.when(base + CHUNK <= N_NODES)
        def _():
            pltpu.sync_copy(acc.at[pl.ds(base, CHUNK)], sbuf0)
            pltpu.sync_copy(sbuf0, out_h.at[pl.ds(base, CHUNK), csl])

        tail = N_NODES % CHUNK

        @pl.when((base < N_NODES) & (base + CHUNK > N_NODES))
        def _():
            pltpu.sync_copy(acc.at[pl.ds(base, tail)],
                            sbuf0.at[pl.ds(0, tail)])
            pltpu.sync_copy(sbuf0.at[pl.ds(0, tail)],
                            out_h.at[pl.ds(base, tail), csl])


def kernel(x, edge_index, edge_values):
    row = edge_index[0].astype(jnp.int32)
    col = edge_index[1].astype(jnp.int32)
    vals = edge_values.astype(jnp.float32)
    pad = N_EDGES_PAD - N_EDGES
    shape3 = (NUM_SUBCORES, CHUNKS_PER_TILE, CHUNK)
    row_p = jnp.pad(row, (0, pad)).reshape(shape3)
    col_p = jnp.pad(col, (0, pad)).reshape(shape3)
    val_p = jnp.pad(vals, (0, pad)).reshape(shape3)
    xb = x.astype(jnp.bfloat16)[:, _PERM]
    return _spmm_sc(xb, col_p, row_p, val_p)


# speedup vs baseline: 1.0955x; 1.0955x over previous
import functools

import jax
import jax.numpy as jnp
from jax import lax
from jax.experimental import pallas as pl
from jax.experimental.pallas import tpu as pltpu
from jax.experimental.pallas import tpu_sc as plsc

N_NODES = 10000
N_EDGES = 320000
D_FEAT = 128
D_HALF = D_FEAT // 2

NUM_SUBCORES = 16
CHUNK = 128
CHUNKS_PER_TILE = 160
EDGES_PER_TILE = CHUNK * CHUNKS_PER_TILE
N_EDGES_PAD = EDGES_PER_TILE * NUM_SUBCORES
N_NODES_PAD = 10240
ROWS_PER_TILE = N_NODES_PAD // NUM_SUBCORES
X_ROWS_PER_TILE = N_NODES // NUM_SUBCORES
NBUF = 2
PHASES = 4
PCH = CHUNKS_PER_TILE // PHASES

_mesh = plsc.VectorSubcoreMesh(core_axis_name="c", subcore_axis_name="s")


@functools.partial(
    pl.kernel,
    out_type=jax.ShapeDtypeStruct((N_NODES, D_FEAT), jnp.float32),
    mesh=_mesh,
    compiler_params=pltpu.CompilerParams(use_tc_tiling_on_sc=False),
    scratch_types=[
        pltpu.VMEM((PCH, CHUNK), jnp.int32),
        pltpu.VMEM((PCH, CHUNK), jnp.int32),
        pltpu.VMEM((PCH, CHUNK), jnp.float32),
        pltpu.VMEM((CHUNK, D_HALF), jnp.float32),
        pltpu.VMEM((CHUNK, D_HALF), jnp.float32),
        pltpu.VMEM((CHUNK, D_HALF), jnp.float32),
        pltpu.VMEM((CHUNK, D_HALF), jnp.float32),
        pltpu.VMEM_SHARED((N_NODES, D_HALF), jnp.float32),
        pltpu.VMEM_SHARED((N_NODES_PAD, D_HALF), jnp.float32),
        pltpu.SemaphoreType.DMA,
        pltpu.SemaphoreType.DMA,
        pltpu.SemaphoreType.DMA,
        pltpu.SemaphoreType.DMA,
    ],
)
def _spmm_sc(xh0_h, xh1_h, col3_h, row3_h, val3_h, out_h,
             colv, rowv, valv, gbuf0, gbuf1, sbuf0, sbuf1, xs, acc,
             gsem0, gsem1, ssem0, ssem1):
    c = lax.axis_index("c")
    s = lax.axis_index("s")
    gbuf = (gbuf0, gbuf1)
    sbuf = (sbuf0, sbuf1)
    gsem = (gsem0, gsem1)
    ssem = (ssem0, ssem1)

    xsl = pl.ds(s * X_ROWS_PER_TILE, X_ROWS_PER_TILE)

    @pl.when(c == 0)
    def _():
        pltpu.async_copy(xh0_h.at[xsl], xs.at[xsl], gsem0)

    @pl.when(c == 1)
    def _():
        pltpu.async_copy(xh1_h.at[xsl], xs.at[xsl], gsem0)

    def zero_body(i, carry):
        for f in range(D_HALF // 16):
            gbuf0[i, pl.ds(f * 16, 16)] = jnp.zeros((16,), jnp.float32)
        return carry
    lax.fori_loop(0, CHUNK, zero_body, 0)
    for i in range(ROWS_PER_TILE // CHUNK):
        pltpu.sync_copy(
            gbuf0, acc.at[pl.ds(s * ROWS_PER_TILE + i * CHUNK, CHUNK)])
    pltpu.make_async_copy(xh0_h.at[xsl], xs.at[xsl], gsem0).wait()
    plsc.subcore_barrier()

    def gather_start(u, b):
        hbm = lax.rem(u, 5) < 2

        @pl.when(hbm & (c == 0))
        def _():
            pltpu.async_copy(xh0_h.at[colv.at[u]], gbuf[b], gsem[b])

        @pl.when(hbm & (c == 1))
        def _():
            pltpu.async_copy(xh1_h.at[colv.at[u]], gbuf[b], gsem[b])

        @pl.when(jnp.logical_not(hbm))
        def _():
            pltpu.async_copy(xs.at[colv.at[u]], gbuf[b], gsem[b])

    def gather_wait(u, b):
        pltpu.make_async_copy(xs.at[colv.at[u]], gbuf[b], gsem[b]).wait()

    def scatter_start(u, b):
        pltpu.async_copy(sbuf[b], acc.at[rowv.at[u]], ssem[b], add=True)

    def scatter_wait(u, b):
        pltpu.make_async_copy(sbuf[b], acc.at[rowv.at[u]], ssem[b]).wait()

    for h in range(PHASES):
        k_lo = h * PCH
        pltpu.sync_copy(col3_h.at[s, pl.ds(k_lo, PCH)], colv)
        pltpu.sync_copy(row3_h.at[s, pl.ds(k_lo, PCH)], rowv)
        pltpu.sync_copy(val3_h.at[s, pl.ds(k_lo, PCH)], valv)

        for b in range(NBUF):
            gather_start(b, b)

        def outer_body(o, carry):
            for b in range(NBUF):
                u = o * NBUF + b
                gather_wait(u, b)

                @pl.when(u >= NBUF)
                def _():
                    scatter_wait(u, b)

                def scale_body(g, inner):
                    vv = valv[u, pl.ds(g * 16, 16)]
                    for j in range(16):
                        e = g * 16 + j
                        v = vv[j]
                        for f in range(D_HALF // 16):
                            sl = pl.ds(f * 16, 16)
                            sbuf[b][e, sl] = gbuf[b][e, sl] * v
                    return inner
                lax.fori_loop(0, CHUNK // 16, scale_body, 0)

                scatter_start(u, b)

                un = u + NBUF

                @pl.when(un < PCH)
                def _():
                    gather_start(un, b)
            return carry

        lax.fori_loop(0, PCH // NBUF, outer_body, 0)

        for b in range(NBUF):
            scatter_wait(0, b)

    plsc.subcore_barrier()

    csl = pl.ds(c * D_HALF, D_HALF)
    for i in range(ROWS_PER_TILE // CHUNK):
        base = s * ROWS_PER_TILE + i * CHUNK

        @pl.when(base + CHUNK <= N_NODES)
        def _():
            pltpu.sync_copy(acc.at[pl.ds(base, CHUNK)], gbuf0)
            pltpu.sync_copy(gbuf0, out_h.at[pl.ds(base, CHUNK), csl])

        tail = N_NODES % CHUNK

        @pl.when((base < N_NODES) & (base + CHUNK > N_NODES))
        def _():
            pltpu.sync_copy(acc.at[pl.ds(base, tail)], gbuf0.at[pl.ds(0, tail)])
            pltpu.sync_copy(gbuf0.at[pl.ds(0, tail)],
                            out_h.at[pl.ds(base, tail), csl])


def kernel(x, edge_index, edge_values):
    row = edge_index[0].astype(jnp.int32)
    col = edge_index[1].astype(jnp.int32)
    vals = edge_values.astype(jnp.float32)
    pad = N_EDGES_PAD - N_EDGES
    shape3 = (NUM_SUBCORES, CHUNKS_PER_TILE, CHUNK)
    row_p = jnp.pad(row, (0, pad)).reshape(shape3)
    col_p = jnp.pad(col, (0, pad)).reshape(shape3)
    val_p = jnp.pad(vals, (0, pad)).reshape(shape3)
    return _spmm_sc(x[:, :D_HALF], x[:, D_HALF:], col_p, row_p, val_p)


# speedup vs baseline: 1.6336x; 1.4911x over previous
import functools

import jax
import jax.numpy as jnp
from jax import lax
from jax.experimental import pallas as pl
from jax.experimental.pallas import tpu as pltpu
from jax.experimental.pallas import tpu_sc as plsc

N_NODES = 10000
N_EDGES = 320000
D_FEAT = 128
D_HALF = D_FEAT // 2

NUM_SUBCORES = 16
CHUNK = 128
CHUNKS_PER_TILE = 160
EDGES_PER_TILE = CHUNK * CHUNKS_PER_TILE
N_EDGES_PAD = EDGES_PER_TILE * NUM_SUBCORES
N_NODES_PAD = 10240
ROWS_PER_TILE = N_NODES_PAD // NUM_SUBCORES
X_ROWS_PER_TILE = N_NODES // NUM_SUBCORES
NBUF = 2
PHASES = 4
PCH = CHUNKS_PER_TILE // PHASES

_mesh = plsc.VectorSubcoreMesh(core_axis_name="c", subcore_axis_name="s")


@functools.partial(
    pl.kernel,
    out_type=jax.ShapeDtypeStruct((N_NODES, D_FEAT), jnp.float32),
    mesh=_mesh,
    compiler_params=pltpu.CompilerParams(use_tc_tiling_on_sc=False),
    scratch_types=[
        pltpu.VMEM((PCH, CHUNK), jnp.int32),
        pltpu.VMEM((PCH, CHUNK), jnp.int32),
        pltpu.VMEM((PCH, CHUNK), jnp.float32),
        pltpu.VMEM((CHUNK, D_HALF), jnp.float32),
        pltpu.VMEM((CHUNK, D_HALF), jnp.float32),
        pltpu.VMEM((CHUNK, D_HALF), jnp.float32),
        pltpu.VMEM((CHUNK, D_HALF), jnp.float32),
        pltpu.VMEM_SHARED((N_NODES, D_HALF), jnp.float32),
        pltpu.VMEM_SHARED((N_NODES_PAD, D_HALF), jnp.float32),
        pltpu.SemaphoreType.DMA,
        pltpu.SemaphoreType.DMA,
        pltpu.SemaphoreType.DMA,
        pltpu.SemaphoreType.DMA,
    ],
)
def _spmm_sc(x_h, col3_h, row3_h, val3_h, out_h,
             colv, rowv, valv, gbuf0, gbuf1, sbuf0, sbuf1, xs, acc,
             gsem0, gsem1, ssem0, ssem1):
    c = lax.axis_index("c")
    s = lax.axis_index("s")
    gbuf = (gbuf0, gbuf1)
    sbuf = (sbuf0, sbuf1)
    gsem = (gsem0, gsem1)
    ssem = (ssem0, ssem1)

    xsl = pl.ds(s * X_ROWS_PER_TILE, X_ROWS_PER_TILE)
    pltpu.async_copy(x_h.at[xsl, pl.ds(c * D_HALF, D_HALF)], xs.at[xsl],
                     gsem0)

    def zero_body(i, carry):
        for f in range(D_HALF // 16):
            gbuf0[i, pl.ds(f * 16, 16)] = jnp.zeros((16,), jnp.float32)
        return carry
    lax.fori_loop(0, CHUNK, zero_body, 0)
    for i in range(ROWS_PER_TILE // CHUNK):
        pltpu.sync_copy(
            gbuf0, acc.at[pl.ds(s * ROWS_PER_TILE + i * CHUNK, CHUNK)])
    pltpu.make_async_copy(x_h.at[xsl, pl.ds(c * D_HALF, D_HALF)], xs.at[xsl],
                          gsem0).wait()
    plsc.subcore_barrier()

    def gather_start(u, b):
        pltpu.async_copy(xs.at[colv.at[u]], gbuf[b], gsem[b])

    def gather_wait(u, b):
        pltpu.make_async_copy(xs.at[colv.at[u]], gbuf[b], gsem[b]).wait()

    def scatter_start(u, b):
        pltpu.async_copy(sbuf[b], acc.at[rowv.at[u]], ssem[b], add=True)

    def scatter_wait(u, b):
        pltpu.make_async_copy(sbuf[b], acc.at[rowv.at[u]], ssem[b]).wait()

    for h in range(PHASES):
        k_lo = h * PCH
        pltpu.async_copy(col3_h.at[s, pl.ds(k_lo, PCH)], colv, ssem0)
        pltpu.async_copy(row3_h.at[s, pl.ds(k_lo, PCH)], rowv, ssem0)
        pltpu.async_copy(val3_h.at[s, pl.ds(k_lo, PCH)], valv, ssem0)
        pltpu.make_async_copy(col3_h.at[s, pl.ds(k_lo, PCH)], colv,
                              ssem0).wait()
        pltpu.make_async_copy(row3_h.at[s, pl.ds(k_lo, PCH)], rowv,
                              ssem0).wait()
        pltpu.make_async_copy(val3_h.at[s, pl.ds(k_lo, PCH)], valv,
                              ssem0).wait()

        for b in range(NBUF):
            gather_start(b, b)

        def outer_body(o, carry):
            for b in range(NBUF):
                u = o * NBUF + b
                gather_wait(u, b)

                @pl.when(u >= NBUF)
                def _():
                    scatter_wait(u, b)

                def scale_body(g, inner):
                    vv = valv[u, pl.ds(g * 16, 16)]
                    for j in range(16):
                        e = g * 16 + j
                        v = vv[j]
                        for f in range(D_HALF // 16):
                            sl = pl.ds(f * 16, 16)
                            sbuf[b][e, sl] = gbuf[b][e, sl] * v
                    return inner
                lax.fori_loop(0, CHUNK // 16, scale_body, 0)

                scatter_start(u, b)

                un = u + NBUF

                @pl.when(un < PCH)
                def _():
                    gather_start(un, b)
            return carry

        lax.fori_loop(0, PCH // NBUF, outer_body, 0)

        for b in range(NBUF):
            scatter_wait(0, b)

    plsc.subcore_barrier()

    csl = pl.ds(c * D_HALF, D_HALF)
    for i in range(ROWS_PER_TILE // CHUNK):
        base = s * ROWS_PER_TILE + i * CHUNK

        @pl.when(base + CHUNK <= N_NODES)
        def _():
            pltpu.sync_copy(acc.at[pl.ds(base, CHUNK)], gbuf0)
            pltpu.sync_copy(gbuf0, out_h.at[pl.ds(base, CHUNK), csl])

        tail = N_NODES % CHUNK

        @pl.when((base < N_NODES) & (base + CHUNK > N_NODES))
        def _():
            pltpu.sync_copy(acc.at[pl.ds(base, tail)], gbuf0.at[pl.ds(0, tail)])
            pltpu.sync_copy(gbuf0.at[pl.ds(0, tail)],
                            out_h.at[pl.ds(base, tail), csl])


def kernel(x, edge_index, edge_values):
    row = edge_index[0].astype(jnp.int32)
    col = edge_index[1].astype(jnp.int32)
    vals = edge_values.astype(jnp.float32)
    pad = N_EDGES_PAD - N_EDGES
    shape3 = (NUM_SUBCORES, CHUNKS_PER_TILE, CHUNK)
    row_p = jnp.pad(row, (0, pad)).reshape(shape3)
    col_p = jnp.pad(col, (0, pad)).reshape(shape3)
    val_p = jnp.pad(vals, (0, pad)).reshape(shape3)
    return _spmm_sc(x, col_p, row_p, val_p)
